# baseline (speedup 1.0000x reference)
import jax
import jax.numpy as jnp
from jax import lax
from jax.experimental import pallas as pl
from jax.experimental.pallas import tpu as pltpu
from jax.experimental.pallas import tpu_sc as plsc

NUM_USERS = 1000000
NUM_ITEMS = 1000
EMBED_DIM = 64
BATCH = 16384
L = 16
PHASE = 128
BU = 32768
QB_SHIFT = (BU // 4).bit_length() - 1


def _sqrt(x):
    bits = plsc.bitcast(x, jnp.int32)
    r = plsc.bitcast(jnp.int32(0x5F3759DF) - (bits >> 1), jnp.float32)
    for _ in range(3):
        r = r * (1.5 - 0.5 * x * r * r)
    return x * r


def _log1p(t):
    s = t / (2.0 + t)
    s2 = s * s
    p = 1.0 / 7.0 + s2 * (1.0 / 9.0)
    p = 1.0 / 5.0 + s2 * p
    p = 1.0 / 3.0 + s2 * p
    return 2.0 * s * (1.0 + s2 * p)


def _pack_body(uet_ref, ubt_ref, uep_ref, ub_ref):
    x = uet_ref[...]
    eye = (lax.broadcasted_iota(jnp.int32, (EMBED_DIM, EMBED_DIM), 0) ==
           lax.broadcasted_iota(jnp.int32, (EMBED_DIM, EMBED_DIM), 1)
           ).astype(jnp.float32)
    xt = lax.dot_general(x, eye, (((0,), (0,)), ((), ())),
                         preferred_element_type=jnp.float32)
    xb = lax.bitcast_convert_type(xt.astype(jnp.bfloat16), jnp.uint16)
    lo = xb[:, :EMBED_DIM // 2].astype(jnp.uint32)
    hi = xb[:, EMBED_DIM // 2:].astype(jnp.uint32) << 16
    w = lax.bitcast_convert_type(lo | hi, jnp.int32)
    q = BU // 4
    uep_ref[...] = jnp.concatenate(
        [w[:q], w[q:2 * q], w[2 * q:3 * q], w[3 * q:]], axis=1)
    ub_ref[...] = jnp.squeeze(ubt_ref[...], axis=0)


def _pack(user_e, user_b):
    grid = (NUM_USERS + BU - 1) // BU
    return pl.pallas_call(
        _pack_body,
        grid=(grid,),
        in_specs=[
            pl.BlockSpec((EMBED_DIM, BU), lambda j: (0, j)),
            pl.BlockSpec((1, BU), lambda j: (0, j)),
        ],
        out_specs=[
            pl.BlockSpec((BU // 4, 2 * EMBED_DIM), lambda j: (j, 0)),
            pl.BlockSpec((BU,), lambda j: (j,)),
        ],
        out_shape=[
            jax.ShapeDtypeStruct((grid * (BU // 4), 2 * EMBED_DIM),
                                 jnp.int32),
            jax.ShapeDtypeStruct((grid * BU,), jnp.float32),
        ],
    )(user_e.T, user_b.T)


def _pack_items(item_e_c):
    xb = lax.bitcast_convert_type(item_e_c.astype(jnp.bfloat16), jnp.uint16)
    lo = xb[:, :EMBED_DIM // 2].astype(jnp.uint32)
    hi = xb[:, EMBED_DIM // 2:].astype(jnp.uint32) << 16
    w = lax.bitcast_convert_type(lo | hi, jnp.int32)
    return w.reshape(NUM_ITEMS // 4, 2 * EMBED_DIM)


def _tile_body(user_ref, item_ref, urow_ref, irow_ref, uep_ref, iep_ref,
               ub_ref, ib_ref, pop_ref, out_ref,
               uidx, iidx, urowi, irowi, urows, irows, ubv, ibv, wpv, outv,
               sem_a, sem_b, sem_s):
    info = plsc.get_sparse_core_info()
    nc = info.num_cores
    wid = lax.axis_index("s") * nc + lax.axis_index("c")
    b_per_w = BATCH // (nc * info.num_subcores)
    n_phases = b_per_w // PHASE
    sems = [sem_a, sem_b]

    pltpu.sync_copy(user_ref.at[wid], uidx)
    pltpu.sync_copy(item_ref.at[wid], iidx)
    pltpu.sync_copy(urow_ref.at[wid], urowi)
    pltpu.sync_copy(irow_ref.at[wid], irowi)

    small = []
    for j in range(n_phases):
        sl = pl.ds(j * PHASE, PHASE)
        small.append(pltpu.async_copy(ub_ref.at[uidx.at[sl]], ubv.at[sl], sem_s))
        small.append(pltpu.async_copy(ib_ref.at[iidx.at[sl]], ibv.at[sl], sem_s))
        small.append(pltpu.async_copy(pop_ref.at[iidx.at[sl]], wpv.at[sl], sem_s))

    def fire_rows(phase):
        b = phase % 2
        sl = pl.ds(phase * PHASE, PHASE)
        return [
            pltpu.async_copy(uep_ref.at[urowi.at[sl]], urows.at[b], sems[b]),
            pltpu.async_copy(iep_ref.at[irowi.at[sl]], irows.at[b], sems[b]),
        ]

    rows = fire_rows(0)
    for c in small:
        c.wait()

    for phase in range(n_phases):
        for c in rows:
            c.wait()
        if phase + 1 < n_phases:
            rows = fire_rows(phase + 1)
        b = phase % 2

        def group(g, _):
            g16 = pl.multiple_of(g * L, L)
            q16 = phase * PHASE + g16
            eids = g16 + lax.iota(jnp.int32, L)
            ucol = ((uidx[pl.ds(q16, L)] >> QB_SHIFT) & 3) * (EMBED_DIM // 2)
            icol = (iidx[pl.ds(q16, L)] & 3) * (EMBED_DIM // 2)
            accs = [jnp.zeros((L,), jnp.float32) for _ in range(4)]
            mask_hi = jnp.int32(-65536)
            for d in range(EMBED_DIM // 2):
                uw = plsc.load_gather(urows.at[b], [eids, ucol + d])
                iw = plsc.load_gather(irows.at[b], [eids, icol + d])
                ua = plsc.bitcast(uw << 16, jnp.float32)
                ub2 = plsc.bitcast(uw & mask_hi, jnp.float32)
                ia = plsc.bitcast(iw << 16, jnp.float32)
                ib2 = plsc.bitcast(iw & mask_hi, jnp.float32)
                accs[(2 * d) % 4] = accs[(2 * d) % 4] + ua * ia
                accs[(2 * d + 1) % 4] = accs[(2 * d + 1) % 4] + ub2 * ib2
            acc = (accs[0] + accs[1]) + (accs[2] + accs[3])
            pred = jnp.where(acc <= 0.0, jnp.exp(acc), acc + 1.0)
            p = pred * _sqrt(wpv[pl.ds(q16, L)])
            ls = jnp.minimum(p, 0.0) - _log1p(jnp.exp(-jnp.abs(p)))
            outv[pl.ds(q16, L)] = (
                ls + ubv[pl.ds(q16, L)] + ibv[pl.ds(q16, L)])
            return 0

        lax.fori_loop(0, PHASE // L, group, 0)

    pltpu.sync_copy(outv, out_ref.at[pl.ds(wid * b_per_w, b_per_w)])


def kernel(user, item, user_e, item_e_c, user_b, item_b, pop_item):
    info = plsc.get_sparse_core_info()
    nw = info.num_cores * info.num_subcores
    b_per_w = BATCH // nw

    user = user.astype(jnp.int32)
    item = item.astype(jnp.int32)
    uep, ub1 = _pack(user_e, user_b)

    mesh = plsc.VectorSubcoreMesh(core_axis_name="c", subcore_axis_name="s")
    run = pl.kernel(
        _tile_body,
        mesh=mesh,
        compiler_params=pltpu.CompilerParams(needs_layout_passes=False,
                                             use_tc_tiling_on_sc=True),
        out_type=jax.ShapeDtypeStruct((BATCH,), jnp.float32),
        scratch_types=[
            pltpu.VMEM((b_per_w,), jnp.int32),
            pltpu.VMEM((b_per_w,), jnp.int32),
            pltpu.VMEM((b_per_w,), jnp.int32),
            pltpu.VMEM((b_per_w,), jnp.int32),
            pltpu.VMEM((2, PHASE, 2 * EMBED_DIM), jnp.int32),
            pltpu.VMEM((2, PHASE, 2 * EMBED_DIM), jnp.int32),
            pltpu.VMEM((b_per_w,), jnp.float32),
            pltpu.VMEM((b_per_w,), jnp.float32),
            pltpu.VMEM((b_per_w,), jnp.float32),
            pltpu.VMEM((b_per_w,), jnp.float32),
            pltpu.SemaphoreType.DMA,
            pltpu.SemaphoreType.DMA,
            pltpu.SemaphoreType.DMA,
        ],
    )
    return run(
        user.reshape(nw, b_per_w),
        item.reshape(nw, b_per_w),
        ((user >> (QB_SHIFT + 2)) * (BU // 4)
         + (user & (BU // 4 - 1))).reshape(nw, b_per_w),
        (item >> 2).reshape(nw, b_per_w),
        uep,
        _pack_items(item_e_c),
        ub1,
        item_b.reshape(NUM_ITEMS),
        pop_item,
    )


# speedup vs baseline: 1.2975x; 1.2975x over previous
import jax
import jax.numpy as jnp
from jax import lax
from jax.experimental import pallas as pl
from jax.experimental.pallas import tpu as pltpu
from jax.experimental.pallas import tpu_sc as plsc

NUM_USERS = 1000000
NUM_ITEMS = 1000
EMBED_DIM = 64
BATCH = 16384
L = 16
PHASE = 128
BU = 32768
HB_SHIFT = (BU // 2).bit_length() - 1


def _sqrt(x):
    bits = plsc.bitcast(x, jnp.int32)
    r = plsc.bitcast(jnp.int32(0x5F3759DF) - (bits >> 1), jnp.float32)
    for _ in range(3):
        r = r * (1.5 - 0.5 * x * r * r)
    return x * r


def _log1p(t):
    s = t / (2.0 + t)
    s2 = s * s
    p = 1.0 / 7.0 + s2 * (1.0 / 9.0)
    p = 1.0 / 5.0 + s2 * p
    p = 1.0 / 3.0 + s2 * p
    return 2.0 * s * (1.0 + s2 * p)


def _pack_body(uet_ref, ubt_ref, uep_ref, ub_ref):
    x = uet_ref[...]
    eye = (lax.broadcasted_iota(jnp.int32, (EMBED_DIM, EMBED_DIM), 0) ==
           lax.broadcasted_iota(jnp.int32, (EMBED_DIM, EMBED_DIM), 1)
           ).astype(jnp.float32)
    xt = lax.dot_general(x, eye, (((0,), (0,)), ((), ())),
                         preferred_element_type=jnp.float32)
    uep_ref[...] = jnp.concatenate([xt[:BU // 2], xt[BU // 2:]], axis=1)
    ub_ref[...] = jnp.squeeze(ubt_ref[...], axis=0)


def _pack(user_e, user_b):
    grid = (NUM_USERS + BU - 1) // BU
    return pl.pallas_call(
        _pack_body,
        grid=(grid,),
        in_specs=[
            pl.BlockSpec((EMBED_DIM, BU), lambda j: (0, j)),
            pl.BlockSpec((1, BU), lambda j: (0, j)),
        ],
        out_specs=[
            pl.BlockSpec((BU // 2, 2 * EMBED_DIM), lambda j: (j, 0)),
            pl.BlockSpec((BU,), lambda j: (j,)),
        ],
        out_shape=[
            jax.ShapeDtypeStruct((grid * (BU // 2), 2 * EMBED_DIM),
                                 jnp.float32),
            jax.ShapeDtypeStruct((grid * BU,), jnp.float32),
        ],
    )(user_e.T, user_b.T)


def _tile_body(user_ref, item_ref, urow_ref, irow_ref, uep_ref, iep_ref,
               ub_ref, ib_ref, pop_ref, out_ref,
               uidx, iidx, urowi, irowi, urows, irows, ubv, ibv, wpv, outv,
               sem_a, sem_b, sem_s):
    info = plsc.get_sparse_core_info()
    nc = info.num_cores
    wid = lax.axis_index("s") * nc + lax.axis_index("c")
    b_per_w = BATCH // (nc * info.num_subcores)
    n_phases = b_per_w // PHASE
    sems = [sem_a, sem_b]

    pltpu.sync_copy(user_ref.at[wid], uidx)
    pltpu.sync_copy(item_ref.at[wid], iidx)
    pltpu.sync_copy(urow_ref.at[wid], urowi)
    pltpu.sync_copy(irow_ref.at[wid], irowi)

    small = []
    for j in range(n_phases):
        sl = pl.ds(j * PHASE, PHASE)
        small.append(pltpu.async_copy(ub_ref.at[uidx.at[sl]], ubv.at[sl], sem_s))
        small.append(pltpu.async_copy(ib_ref.at[iidx.at[sl]], ibv.at[sl], sem_s))
        small.append(pltpu.async_copy(pop_ref.at[iidx.at[sl]], wpv.at[sl], sem_s))

    def fire_rows(phase):
        b = phase % 2
        sl = pl.ds(phase * PHASE, PHASE)
        return [
            pltpu.async_copy(uep_ref.at[urowi.at[sl]], urows.at[b], sems[b]),
            pltpu.async_copy(iep_ref.at[irowi.at[sl]], irows.at[b], sems[b]),
        ]

    rows = fire_rows(0)
    for c in small:
        c.wait()

    for phase in range(n_phases):
        for c in rows:
            c.wait()
        if phase + 1 < n_phases:
            rows = fire_rows(phase + 1)
        b = phase % 2

        def group(g, _):
            g16 = pl.multiple_of(g * L, L)
            q16 = phase * PHASE + g16
            eids = g16 + lax.iota(jnp.int32, L)
            ucol = ((uidx[pl.ds(q16, L)] >> HB_SHIFT) & 1) * EMBED_DIM
            icol = (iidx[pl.ds(q16, L)] & 1) * EMBED_DIM
            accs = [jnp.zeros((L,), jnp.float32) for _ in range(4)]
            for d in range(EMBED_DIM):
                uv = plsc.load_gather(urows.at[b], [eids, ucol + d])
                iv = plsc.load_gather(irows.at[b], [eids, icol + d])
                accs[d % 4] = accs[d % 4] + uv * iv
            acc = (accs[0] + accs[1]) + (accs[2] + accs[3])
            pred = jnp.where(acc <= 0.0, jnp.exp(acc), acc + 1.0)
            p = pred * _sqrt(wpv[pl.ds(q16, L)])
            ls = jnp.minimum(p, 0.0) - _log1p(jnp.exp(-jnp.abs(p)))
            outv[pl.ds(q16, L)] = (
                ls + ubv[pl.ds(q16, L)] + ibv[pl.ds(q16, L)])
            return 0

        lax.fori_loop(0, PHASE // L, group, 0)

    pltpu.sync_copy(outv, out_ref.at[pl.ds(wid * b_per_w, b_per_w)])


def kernel(user, item, user_e, item_e_c, user_b, item_b, pop_item):
    info = plsc.get_sparse_core_info()
    nw = info.num_cores * info.num_subcores
    b_per_w = BATCH // nw

    user = user.astype(jnp.int32)
    item = item.astype(jnp.int32)
    uep, ub1 = _pack(user_e, user_b)

    mesh = plsc.VectorSubcoreMesh(core_axis_name="c", subcore_axis_name="s")
    run = pl.kernel(
        _tile_body,
        mesh=mesh,
        compiler_params=pltpu.CompilerParams(needs_layout_passes=False,
                                             use_tc_tiling_on_sc=True),
        out_type=jax.ShapeDtypeStruct((BATCH,), jnp.float32),
        scratch_types=[
            pltpu.VMEM((b_per_w,), jnp.int32),
            pltpu.VMEM((b_per_w,), jnp.int32),
            pltpu.VMEM((b_per_w,), jnp.int32),
            pltpu.VMEM((b_per_w,), jnp.int32),
            pltpu.VMEM((2, PHASE, 2 * EMBED_DIM), jnp.float32),
            pltpu.VMEM((2, PHASE, 2 * EMBED_DIM), jnp.float32),
            pltpu.VMEM((b_per_w,), jnp.float32),
            pltpu.VMEM((b_per_w,), jnp.float32),
            pltpu.VMEM((b_per_w,), jnp.float32),
            pltpu.VMEM((b_per_w,), jnp.float32),
            pltpu.SemaphoreType.DMA,
            pltpu.SemaphoreType.DMA,
            pltpu.SemaphoreType.DMA,
        ],
    )
    return run(
        user.reshape(nw, b_per_w),
        item.reshape(nw, b_per_w),
        ((user >> (HB_SHIFT + 1)) * (BU // 2)
         + (user & (BU // 2 - 1))).reshape(nw, b_per_w),
        (item >> 1).reshape(nw, b_per_w),
        uep,
        item_e_c.reshape(NUM_ITEMS // 2, 2 * EMBED_DIM),
        ub1,
        item_b.reshape(NUM_ITEMS),
        pop_item,
    )
